# Initial kernel scaffold; baseline (speedup 1.0000x reference)
import functools

import jax
import jax.numpy as jnp
from jax import lax
from jax.experimental import pallas as pl
from jax.experimental.pallas import tpu as pltpu
from jax.experimental.pallas import tpu_sc as plsc


_B, _C, _H, _W = 2, 96, 384, 384
_HW = _H * _W
_N = _B * _HW
_NW = 32
_RPW = _N // _NW
_CH = 64
_NCH = _RPW // _CH
_BR = 96


def _index_weight_kernel(aff_ref, ia, ib, ic, id_, wa, wb, wc, wd):
    b = pl.program_id(0)
    blk = pl.program_id(1)
    shape = (1, _BR, _H)
    ix = lax.broadcasted_iota(jnp.float32, shape, 1) + (blk * _BR).astype(
        jnp.float32
    )
    iy = lax.broadcasted_iota(jnp.float32, shape, 2)
    step_x = jnp.float32(2.0 / (_W - 1))
    step_y = jnp.float32(2.0 / (_H - 1))
    xc = ix * step_x - 1.0
    yc = iy * step_y - 1.0

    t00 = aff_ref[0, 0]
    t01 = aff_ref[0, 1]
    t02 = aff_ref[0, 2]
    t10 = aff_ref[1, 0]
    t11 = aff_ref[1, 1]
    t12 = aff_ref[1, 2]
    t20 = aff_ref[2, 0]
    t21 = aff_ref[2, 1]
    t22 = aff_ref[2, 2]

    sx = xc * t00 + yc * t10 + t20
    sy = xc * t01 + yc * t11 + t21
    sz = xc * t02 + yc * t12 + t22
    w = 1.0 / sz
    x = 0.5 * (sx * w + 1.0) * _W
    y = 0.5 * (sy * w + 1.0) * _H

    x0 = jnp.round(x).astype(jnp.int32)
    x1 = x0 + 1
    y0 = jnp.round(y).astype(jnp.int32)
    y1 = y0 + 1
    x0 = jnp.clip(x0, 0, _W - 1)
    x1 = jnp.clip(x1, 0, _W - 1)
    y0 = jnp.clip(y0, 0, _H - 1)
    y1 = jnp.clip(y1, 0, _H - 1)

    base = b * _HW
    by0 = base + y0 * _W
    by1 = base + y1 * _W
    ia[...] = by0 + x0
    ib[...] = by1 + x0
    ic[...] = by0 + x1
    id_[...] = by1 + x1

    x0f = x0.astype(jnp.float32)
    x1f = x1.astype(jnp.float32)
    y0f = y0.astype(jnp.float32)
    y1f = y1.astype(jnp.float32)
    wa[...] = (x1f - x) * (y1f - y)
    wb[...] = (x1f - x) * (y - y0f)
    wc[...] = (x - x0f) * (y1f - y)
    wd[...] = (x - x0f) * (y - y0f)


def _compute_indices_weights(aff):
    grid = (_B, _W // _BR)
    blk = (1, _BR, _H)
    out_shape = [jax.ShapeDtypeStruct((_B, _W, _H), jnp.int32)] * 4 + [
        jax.ShapeDtypeStruct((_B, _W, _H), jnp.float32)
    ] * 4
    specs = [pl.BlockSpec(blk, lambda b, i: (b, i, 0))] * 8
    return pl.pallas_call(
        _index_weight_kernel,
        grid=grid,
        in_specs=[pl.BlockSpec(memory_space=pltpu.SMEM)],
        out_specs=specs,
        out_shape=out_shape,
    )(aff)


def _sc_gather_combine(table, ia, ib, ic, id_, wa, wb, wc, wd):
    mesh = plsc.VectorSubcoreMesh(core_axis_name="c", subcore_axis_name="s")

    @functools.partial(
        pl.kernel,
        out_type=jax.ShapeDtypeStruct((_N, _C), jnp.float32),
        mesh=mesh,
        scratch_types=[
            pltpu.VMEM((_RPW,), jnp.int32),
            pltpu.VMEM((_RPW,), jnp.int32),
            pltpu.VMEM((_RPW,), jnp.int32),
            pltpu.VMEM((_RPW,), jnp.int32),
            pltpu.VMEM((_RPW,), jnp.float32),
            pltpu.VMEM((_RPW,), jnp.float32),
            pltpu.VMEM((_RPW,), jnp.float32),
            pltpu.VMEM((_RPW,), jnp.float32),
            pltpu.VMEM((_CH, _C), jnp.float32),
            pltpu.VMEM((_CH, _C), jnp.float32),
            pltpu.VMEM((_CH, _C), jnp.float32),
            pltpu.VMEM((_CH, _C), jnp.float32),
            pltpu.VMEM((_CH, _C), jnp.float32),
            pltpu.SemaphoreType.DMA,
        ],
    )
    def run(
        table_hbm, ia_hbm, ib_hbm, ic_hbm, id_hbm, wa_hbm, wb_hbm, wc_hbm,
        wd_hbm, out_hbm, iav, ibv, icv, idv, wav, wbv, wcv, wdv,
        bufa, bufb, bufc, bufd, outv, sem,
    ):
        wid = lax.axis_index("s") * 2 + lax.axis_index("c")
        base = wid * _RPW
        pltpu.sync_copy(ia_hbm.at[pl.ds(base, _RPW)], iav)
        pltpu.sync_copy(ib_hbm.at[pl.ds(base, _RPW)], ibv)
        pltpu.sync_copy(ic_hbm.at[pl.ds(base, _RPW)], icv)
        pltpu.sync_copy(id_hbm.at[pl.ds(base, _RPW)], idv)
        pltpu.sync_copy(wa_hbm.at[pl.ds(base, _RPW)], wav)
        pltpu.sync_copy(wb_hbm.at[pl.ds(base, _RPW)], wbv)
        pltpu.sync_copy(wc_hbm.at[pl.ds(base, _RPW)], wcv)
        pltpu.sync_copy(wd_hbm.at[pl.ds(base, _RPW)], wdv)

        @pl.loop(0, _NCH)
        def chunk_loop(t):
            off = t * _CH
            da = pltpu.async_copy(
                table_hbm.at[iav.at[pl.ds(off, _CH)]], bufa, sem
            )
            db = pltpu.async_copy(
                table_hbm.at[ibv.at[pl.ds(off, _CH)]], bufb, sem
            )
            dc = pltpu.async_copy(
                table_hbm.at[icv.at[pl.ds(off, _CH)]], bufc, sem
            )
            dd = pltpu.async_copy(
                table_hbm.at[idv.at[pl.ds(off, _CH)]], bufd, sem
            )
            da.wait()
            db.wait()
            dc.wait()
            dd.wait()

            @pl.loop(0, _CH)
            def row_loop(r):
                ws_a = wav[off + r]
                ws_b = wbv[off + r]
                ws_c = wcv[off + r]
                ws_d = wdv[off + r]
                for j in range(_C // 16):
                    sl = pl.ds(j * 16, 16)
                    outv[r, sl] = (
                        (bufa[r, sl] * ws_a + bufb[r, sl] * ws_b)
                        + bufc[r, sl] * ws_c
                    ) + bufd[r, sl] * ws_d

            pltpu.sync_copy(outv, out_hbm.at[pl.ds(base + off, _CH)])

    return run(table, ia, ib, ic, id_, wa, wb, wc, wd)


def kernel(x, affine_transformation):
    B, C, H, W = x.shape
    assert (B, C, H, W) == (_B, _C, _H, _W)
    table = x.reshape(-1, C).astype(jnp.uint8).astype(jnp.float32)
    aff = affine_transformation.reshape(3, 3)

    ia, ib, ic, id_, wa, wb, wc, wd = _compute_indices_weights(aff)
    flat = lambda a: a.reshape(_N)
    out = _sc_gather_combine(
        table,
        flat(ia), flat(ib), flat(ic), flat(id_),
        flat(wa), flat(wb), flat(wc), flat(wd),
    )
    return out.reshape(B, C, H, W)


# speedup vs baseline: 9.3935x; 9.3935x over previous
import functools

import jax
import jax.numpy as jnp
from jax import lax
from jax.experimental import pallas as pl
from jax.experimental.pallas import tpu as pltpu
from jax.experimental.pallas import tpu_sc as plsc


_B, _C, _H, _W = 2, 96, 384, 384
_HW = _H * _W
_N = _B * _HW
_NW = 32
_RPW = _N // _NW
_CH = 64
_NCH = _RPW // _CH
_BR = 96


def _index_weight_kernel(aff_ref, ia, ib, ic, id_, wa, wb, wc, wd):
    b = pl.program_id(0)
    blk = pl.program_id(1)
    shape = (1, _BR, _H)
    ix = (
        lax.broadcasted_iota(jnp.int32, shape, 1) + blk * _BR
    ).astype(jnp.float32)
    iy = lax.broadcasted_iota(jnp.int32, shape, 2).astype(jnp.float32)
    step_x = jnp.float32(2.0 / (_W - 1))
    step_y = jnp.float32(2.0 / (_H - 1))
    xc = ix * step_x - 1.0
    yc = iy * step_y - 1.0

    t00 = aff_ref[0, 0]
    t01 = aff_ref[0, 1]
    t02 = aff_ref[0, 2]
    t10 = aff_ref[1, 0]
    t11 = aff_ref[1, 1]
    t12 = aff_ref[1, 2]
    t20 = aff_ref[2, 0]
    t21 = aff_ref[2, 1]
    t22 = aff_ref[2, 2]

    sx = xc * t00 + yc * t10 + t20
    sy = xc * t01 + yc * t11 + t21
    sz = xc * t02 + yc * t12 + t22
    w = 1.0 / sz
    x = 0.5 * (sx * w + 1.0) * _W
    y = 0.5 * (sy * w + 1.0) * _H

    x0 = jnp.round(x).astype(jnp.int32)
    x1 = x0 + 1
    y0 = jnp.round(y).astype(jnp.int32)
    y1 = y0 + 1
    x0 = jnp.clip(x0, 0, _W - 1)
    x1 = jnp.clip(x1, 0, _W - 1)
    y0 = jnp.clip(y0, 0, _H - 1)
    y1 = jnp.clip(y1, 0, _H - 1)

    base = b * _HW
    by0 = base + y0 * _W
    by1 = base + y1 * _W
    ia[...] = by0 + x0
    ib[...] = by1 + x0
    ic[...] = by0 + x1
    id_[...] = by1 + x1

    x0f = x0.astype(jnp.float32)
    x1f = x1.astype(jnp.float32)
    y0f = y0.astype(jnp.float32)
    y1f = y1.astype(jnp.float32)
    wa[...] = (x1f - x) * (y1f - y)
    wb[...] = (x1f - x) * (y - y0f)
    wc[...] = (x - x0f) * (y1f - y)
    wd[...] = (x - x0f) * (y - y0f)


def _compute_indices_weights(aff):
    grid = (_B, _W // _BR)
    blk = (1, _BR, _H)
    out_shape = [jax.ShapeDtypeStruct((_B, _W, _H), jnp.int32)] * 4 + [
        jax.ShapeDtypeStruct((_B, _W, _H), jnp.float32)
    ] * 4
    specs = [pl.BlockSpec(blk, lambda b, i: (b, i, 0))] * 8
    return pl.pallas_call(
        _index_weight_kernel,
        grid=grid,
        in_specs=[pl.BlockSpec(memory_space=pltpu.SMEM)],
        out_specs=specs,
        out_shape=out_shape,
    )(aff)


def _sc_gather_combine(table, ia, ib, ic, id_, wa, wb, wc, wd):
    mesh = plsc.VectorSubcoreMesh(core_axis_name="c", subcore_axis_name="s")

    @functools.partial(
        pl.kernel,
        out_type=jax.ShapeDtypeStruct((_N, _C), jnp.float32),
        mesh=mesh,
        scratch_types=[
            pltpu.VMEM((_RPW,), jnp.int32),
            pltpu.VMEM((_RPW,), jnp.int32),
            pltpu.VMEM((_RPW,), jnp.int32),
            pltpu.VMEM((_RPW,), jnp.int32),
            pltpu.VMEM((_RPW,), jnp.float32),
            pltpu.VMEM((_RPW,), jnp.float32),
            pltpu.VMEM((_RPW,), jnp.float32),
            pltpu.VMEM((_RPW,), jnp.float32),
            pltpu.VMEM((_CH, _C), jnp.float32),
            pltpu.VMEM((_CH, _C), jnp.float32),
            pltpu.VMEM((_CH, _C), jnp.float32),
            pltpu.VMEM((_CH, _C), jnp.float32),
            pltpu.VMEM((_CH, _C), jnp.float32),
            pltpu.SemaphoreType.DMA,
        ],
        compiler_params=pltpu.CompilerParams(use_tc_tiling_on_sc=False),
    )
    def run(
        table_hbm, ia_hbm, ib_hbm, ic_hbm, id_hbm, wa_hbm, wb_hbm, wc_hbm,
        wd_hbm, out_hbm, iav, ibv, icv, idv, wav, wbv, wcv, wdv,
        bufa, bufb, bufc, bufd, outv, sem,
    ):
        wid = lax.axis_index("s") * 2 + lax.axis_index("c")
        base = wid * _RPW
        pltpu.sync_copy(ia_hbm.at[pl.ds(base, _RPW)], iav)
        pltpu.sync_copy(ib_hbm.at[pl.ds(base, _RPW)], ibv)
        pltpu.sync_copy(ic_hbm.at[pl.ds(base, _RPW)], icv)
        pltpu.sync_copy(id_hbm.at[pl.ds(base, _RPW)], idv)
        pltpu.sync_copy(wa_hbm.at[pl.ds(base, _RPW)], wav)
        pltpu.sync_copy(wb_hbm.at[pl.ds(base, _RPW)], wbv)
        pltpu.sync_copy(wc_hbm.at[pl.ds(base, _RPW)], wcv)
        pltpu.sync_copy(wd_hbm.at[pl.ds(base, _RPW)], wdv)

        @pl.loop(0, _NCH)
        def chunk_loop(t):
            off = t * _CH
            da = pltpu.async_copy(
                table_hbm.at[iav.at[pl.ds(off, _CH)]], bufa, sem
            )
            db = pltpu.async_copy(
                table_hbm.at[ibv.at[pl.ds(off, _CH)]], bufb, sem
            )
            dc = pltpu.async_copy(
                table_hbm.at[icv.at[pl.ds(off, _CH)]], bufc, sem
            )
            dd = pltpu.async_copy(
                table_hbm.at[idv.at[pl.ds(off, _CH)]], bufd, sem
            )
            da.wait()
            db.wait()
            dc.wait()
            dd.wait()

            @pl.loop(0, _CH // 16)
            def group_loop(g):
                wva = wav[pl.ds(off + g * 16, 16)]
                wvb = wbv[pl.ds(off + g * 16, 16)]
                wvc = wcv[pl.ds(off + g * 16, 16)]
                wvd = wdv[pl.ds(off + g * 16, 16)]
                for lane in range(16):
                    r = g * 16 + lane
                    ws_a = wva[lane]
                    ws_b = wvb[lane]
                    ws_c = wvc[lane]
                    ws_d = wvd[lane]
                    for j in range(_C // 16):
                        sl = pl.ds(j * 16, 16)
                        outv[r, sl] = (
                            (bufa[r, sl] * ws_a + bufb[r, sl] * ws_b)
                            + bufc[r, sl] * ws_c
                        ) + bufd[r, sl] * ws_d

            pltpu.sync_copy(outv, out_hbm.at[pl.ds(base + off, _CH)])

    return run(table, ia, ib, ic, id_, wa, wb, wc, wd)


def kernel(x, affine_transformation):
    B, C, H, W = x.shape
    assert (B, C, H, W) == (_B, _C, _H, _W)
    table = x.reshape(-1, C).astype(jnp.uint8).astype(jnp.float32)
    aff = affine_transformation.reshape(3, 3)

    ia, ib, ic, id_, wa, wb, wc, wd = _compute_indices_weights(aff)
    flat = lambda a: a.reshape(_N)
    out = _sc_gather_combine(
        table,
        flat(ia), flat(ib), flat(ic), flat(id_),
        flat(wa), flat(wb), flat(wc), flat(wd),
    )
    return out.reshape(B, C, H, W)


# speedup vs baseline: 9.5017x; 1.0115x over previous
import functools

import jax
import jax.numpy as jnp
from jax import lax
from jax.experimental import pallas as pl
from jax.experimental.pallas import tpu as pltpu
from jax.experimental.pallas import tpu_sc as plsc


_B, _C, _H, _W = 2, 96, 384, 384
_HW = _H * _W
_N = _B * _HW
_NW = 32
_RPW = _N // _NW
_CH = 32
_NCH = _RPW // _CH
_BR = 96


def _index_weight_kernel(aff_ref, ia, ib, ic, id_, wa, wb, wc, wd):
    b = pl.program_id(0)
    blk = pl.program_id(1)
    shape = (1, _BR, _H)
    ix = (
        lax.broadcasted_iota(jnp.int32, shape, 1) + blk * _BR
    ).astype(jnp.float32)
    iy = lax.broadcasted_iota(jnp.int32, shape, 2).astype(jnp.float32)
    step_x = jnp.float32(2.0 / (_W - 1))
    step_y = jnp.float32(2.0 / (_H - 1))
    xc = ix * step_x - 1.0
    yc = iy * step_y - 1.0

    t00 = aff_ref[0, 0]
    t01 = aff_ref[0, 1]
    t02 = aff_ref[0, 2]
    t10 = aff_ref[1, 0]
    t11 = aff_ref[1, 1]
    t12 = aff_ref[1, 2]
    t20 = aff_ref[2, 0]
    t21 = aff_ref[2, 1]
    t22 = aff_ref[2, 2]

    sx = xc * t00 + yc * t10 + t20
    sy = xc * t01 + yc * t11 + t21
    sz = xc * t02 + yc * t12 + t22
    w = 1.0 / sz
    x = 0.5 * (sx * w + 1.0) * _W
    y = 0.5 * (sy * w + 1.0) * _H

    x0 = jnp.round(x).astype(jnp.int32)
    x1 = x0 + 1
    y0 = jnp.round(y).astype(jnp.int32)
    y1 = y0 + 1
    x0 = jnp.clip(x0, 0, _W - 1)
    x1 = jnp.clip(x1, 0, _W - 1)
    y0 = jnp.clip(y0, 0, _H - 1)
    y1 = jnp.clip(y1, 0, _H - 1)

    base = b * _HW
    by0 = base + y0 * _W
    by1 = base + y1 * _W
    ia[...] = by0 + x0
    ib[...] = by1 + x0
    ic[...] = by0 + x1
    id_[...] = by1 + x1

    x0f = x0.astype(jnp.float32)
    x1f = x1.astype(jnp.float32)
    y0f = y0.astype(jnp.float32)
    y1f = y1.astype(jnp.float32)
    wa[...] = (x1f - x) * (y1f - y)
    wb[...] = (x1f - x) * (y - y0f)
    wc[...] = (x - x0f) * (y1f - y)
    wd[...] = (x - x0f) * (y - y0f)


def _compute_indices_weights(aff):
    grid = (_B, _W // _BR)
    blk = (1, _BR, _H)
    out_shape = [jax.ShapeDtypeStruct((_B, _W, _H), jnp.int32)] * 4 + [
        jax.ShapeDtypeStruct((_B, _W, _H), jnp.float32)
    ] * 4
    specs = [pl.BlockSpec(blk, lambda b, i: (b, i, 0))] * 8
    return pl.pallas_call(
        _index_weight_kernel,
        grid=grid,
        in_specs=[pl.BlockSpec(memory_space=pltpu.SMEM)],
        out_specs=specs,
        out_shape=out_shape,
    )(aff)


def _sc_gather_combine(table, ia, ib, ic, id_, wa, wb, wc, wd):
    mesh = plsc.VectorSubcoreMesh(core_axis_name="c", subcore_axis_name="s")

    @functools.partial(
        pl.kernel,
        out_type=jax.ShapeDtypeStruct((_N, _C), jnp.float32),
        mesh=mesh,
        scratch_types=[
            pltpu.VMEM((_NCH, _CH), jnp.int32),
            pltpu.VMEM((_NCH, _CH), jnp.int32),
            pltpu.VMEM((_NCH, _CH), jnp.int32),
            pltpu.VMEM((_NCH, _CH), jnp.int32),
            pltpu.VMEM((_RPW,), jnp.float32),
            pltpu.VMEM((_RPW,), jnp.float32),
            pltpu.VMEM((_RPW,), jnp.float32),
            pltpu.VMEM((_RPW,), jnp.float32),
            pltpu.VMEM((_CH, _C), jnp.float32),
            pltpu.VMEM((_CH, _C), jnp.float32),
            pltpu.VMEM((_CH, _C), jnp.float32),
            pltpu.VMEM((_CH, _C), jnp.float32),
            pltpu.VMEM((_CH, _C), jnp.float32),
            pltpu.VMEM((_CH, _C), jnp.float32),
            pltpu.VMEM((_CH, _C), jnp.float32),
            pltpu.VMEM((_CH, _C), jnp.float32),
            pltpu.VMEM((_CH, _C), jnp.float32),
            pltpu.VMEM((_CH, _C), jnp.float32),
            pltpu.VMEM((_CH, _C), jnp.float32),
            pltpu.VMEM((_CH, _C), jnp.float32),
            pltpu.SemaphoreType.DMA,
            pltpu.SemaphoreType.DMA,
            pltpu.SemaphoreType.DMA,
        ],
        compiler_params=pltpu.CompilerParams(use_tc_tiling_on_sc=False),
    )
    def run(
        table_hbm, ia_hbm, ib_hbm, ic_hbm, id_hbm, wa_hbm, wb_hbm, wc_hbm,
        wd_hbm, out_hbm, iav, ibv, icv, idv, wav, wbv, wcv, wdv,
        bufa0, bufb0, bufc0, bufd0, bufa1, bufb1, bufc1, bufd1,
        outv0, outv1, outv2, outv3, gsem0, gsem1, osem,
    ):
        wid = lax.axis_index("s") * 2 + lax.axis_index("c")
        base = wid * _RPW
        crow = wid * _NCH
        pltpu.sync_copy(ia_hbm.at[pl.ds(crow, _NCH)], iav)
        pltpu.sync_copy(ib_hbm.at[pl.ds(crow, _NCH)], ibv)
        pltpu.sync_copy(ic_hbm.at[pl.ds(crow, _NCH)], icv)
        pltpu.sync_copy(id_hbm.at[pl.ds(crow, _NCH)], idv)
        pltpu.sync_copy(wa_hbm.at[pl.ds(base, _RPW)], wav)
        pltpu.sync_copy(wb_hbm.at[pl.ds(base, _RPW)], wbv)
        pltpu.sync_copy(wc_hbm.at[pl.ds(base, _RPW)], wcv)
        pltpu.sync_copy(wd_hbm.at[pl.ds(base, _RPW)], wdv)

        slot_bufs = (
            (bufa0, bufb0, bufc0, bufd0),
            (bufa1, bufb1, bufc1, bufd1),
        )
        gsems = (gsem0, gsem1)
        outvs = (outv0, outv1, outv2, outv3)

        def issue_gathers(t, s):
            return [
                pltpu.async_copy(
                    table_hbm.at[iv.at[t]], buf, gsems[s]
                )
                for iv, buf in zip((iav, ibv, icv, idv), slot_bufs[s])
            ]

        def combine(t, s, outv):
            off = t * _CH
            sl_bufa, sl_bufb, sl_bufc, sl_bufd = slot_bufs[s]

            @pl.loop(0, _CH // 16)
            def group_loop(g):
                wva = wav[pl.ds(off + g * 16, 16)]
                wvb = wbv[pl.ds(off + g * 16, 16)]
                wvc = wcv[pl.ds(off + g * 16, 16)]
                wvd = wdv[pl.ds(off + g * 16, 16)]
                for lane in range(16):
                    r = g * 16 + lane
                    ws_a = wva[lane]
                    ws_b = wvb[lane]
                    ws_c = wvc[lane]
                    ws_d = wvd[lane]
                    for j in range(_C // 16):
                        cs = pl.ds(j * 16, 16)
                        outv[r, cs] = (
                            (sl_bufa[r, cs] * ws_a + sl_bufb[r, cs] * ws_b)
                            + sl_bufc[r, cs] * ws_c
                        ) + sl_bufd[r, cs] * ws_d

            pltpu.sync_copy(outv, out_hbm.at[pl.ds(base + off, _CH)])

        @pl.loop(0, _NCH, step=4)
        def chunk_loop(t0):
            gd = [issue_gathers(t0, 0), issue_gathers(t0 + 1, 1)]
            for k in range(4):
                s = k % 2
                for d in gd[k]:
                    d.wait()
                combine(t0 + k, s, outvs[k])
                if k + 2 < 4:
                    gd.append(issue_gathers(t0 + k + 2, s))

    return run(table, ia, ib, ic, id_, wa, wb, wc, wd)


def kernel(x, affine_transformation):
    B, C, H, W = x.shape
    assert (B, C, H, W) == (_B, _C, _H, _W)
    table = x.reshape(-1, C).astype(jnp.uint8).astype(jnp.float32)
    aff = affine_transformation.reshape(3, 3)

    ia, ib, ic, id_, wa, wb, wc, wd = _compute_indices_weights(aff)
    flat = lambda a: a.reshape(_N)
    rows = lambda a: a.reshape(_N // _CH, _CH)
    out = _sc_gather_combine(
        table,
        rows(ia), rows(ib), rows(ic), rows(id_),
        flat(wa), flat(wb), flat(wc), flat(wd),
    )
    return out.reshape(B, C, H, W)
